# baseline (speedup 1.0000x reference)
import functools

import jax
import jax.numpy as jnp
from jax import lax
from jax.experimental import pallas as pl
from jax.experimental.pallas import tpu as pltpu
from jax.experimental.pallas import tpu_sc as plsc

_NC = 2
_NS = 16
_NW = _NC * _NS
_LANES = 16
_CHUNK = 128


def _perm_flat(x):
    b, m = x.shape
    return x.reshape(b // 128, 128, m // 8, 8).transpose(2, 0, 3, 1).reshape(-1)


def _sc_gather(scores_flat, idx_flat, n_bt):
    total = idx_flat.shape[0]
    assert total % (_NW * _CHUNK) == 0
    per_w = total // _NW
    n_chunks = per_w // _CHUNK
    n_vec = per_w // _LANES

    mesh = plsc.VectorSubcoreMesh(core_axis_name="c", subcore_axis_name="s")

    @functools.partial(
        pl.kernel,
        out_type=jax.ShapeDtypeStruct((total,), jnp.float32),
        mesh=mesh,
        scratch_types=[
            pltpu.VMEM((per_w,), jnp.int32),
            pltpu.VMEM((per_w,), jnp.float32),
            pltpu.SemaphoreType.DMA,
            pltpu.SemaphoreType.DMA,
        ],
    )
    def gather_kernel(scores_hbm, idx_hbm, out_hbm, idx_v, vals_v, sem, sem2):
        wid = lax.axis_index("s") * _NC + lax.axis_index("c")
        elem_base = wid * per_w
        half = per_w // 2
        stage2 = pltpu.make_async_copy(
            idx_hbm.at[pl.ds(elem_base + half, half)],
            idx_v.at[pl.ds(half, half)],
            sem2,
        )
        stage2.start()
        pltpu.sync_copy(idx_hbm.at[pl.ds(elem_base, half)], idx_v.at[pl.ds(0, half)])

        lane = lax.iota(jnp.int32, _LANES)
        sh = 10 + (n_bt.bit_length() - 1)
        m_hi = jnp.int32(-(1 << sh))
        m_jr = jnp.int32(7 << 7)
        m_b = jnp.int32(((n_bt - 1) << 10) | 127)

        def chunk_body(c, _):
            col0 = c * _CHUNK
            for v in range(_CHUNK // _LANES):
                col = col0 + v * _LANES
                q = lax.broadcast(elem_base + col, (_LANES,)) + lane
                j = idx_v[pl.ds(col, _LANES)]
                hi = lax.bitwise_and(lax.shift_left(j, sh - 3), m_hi)
                jr = lax.bitwise_and(lax.shift_left(j, 7), m_jr)
                idx_v[pl.ds(col, _LANES)] = lax.bitwise_or(
                    lax.bitwise_or(hi, jr), lax.bitwise_and(q, m_b)
                )
            pltpu.make_async_copy(
                scores_hbm.at[idx_v.at[pl.ds(col0, _CHUNK)]],
                vals_v.at[pl.ds(col0, _CHUNK)],
                sem,
            ).start()
            return 0

        lax.fori_loop(0, n_chunks // 2, chunk_body, 0)
        stage2.wait()
        lax.fori_loop(n_chunks // 2, n_chunks, chunk_body, 0)
        pltpu.make_async_copy(
            scores_hbm.at[pl.ds(0, per_w)], vals_v, sem
        ).wait()

        pltpu.sync_copy(vals_v, out_hbm.at[pl.ds(elem_base, per_w)])

    return gather_kernel(scores_flat, idx_flat)


def _tc_log_softmax_perm(g4):
    def body(x_ref, o_ref):
        x = x_ref[...]
        m = jnp.max(jnp.max(x, axis=0, keepdims=True), axis=2, keepdims=True)
        e = jnp.exp(x - m)
        s = jnp.sum(jnp.sum(e, axis=0, keepdims=True), axis=2, keepdims=True)
        o_ref[...] = (x - m) - jnp.log(s)

    return pl.pallas_call(
        body,
        out_shape=jax.ShapeDtypeStruct(g4.shape, g4.dtype),
    )(g4)


def kernel(scores, idx):
    b, n = scores.shape
    k = idx.shape[1]
    assert b % 128 == 0 and n % 8 == 0 and k % 8 == 0
    assert (b * k) % (_NW * _CHUNK) == 0
    n_bt = b // 128
    assert n_bt & (n_bt - 1) == 0

    gathered = _sc_gather(_perm_flat(scores), _perm_flat(idx), n_bt)
    out4 = _tc_log_softmax_perm(gathered.reshape(k // 8, n_bt, 8, 128))
    return out4.transpose(1, 3, 0, 2).reshape(b, k)


# speedup vs baseline: 1.0075x; 1.0075x over previous
import functools

import jax
import jax.numpy as jnp
from jax import lax
from jax.experimental import pallas as pl
from jax.experimental.pallas import tpu as pltpu
from jax.experimental.pallas import tpu_sc as plsc

_NC = 2
_NS = 16
_NW = _NC * _NS
_LANES = 16
_CHUNK = 128


def _perm_flat(x):
    b, m = x.shape
    return x.reshape(b // 128, 128, m // 8, 8).transpose(2, 0, 3, 1).reshape(-1)


def _sc_gather(scores_flat, idx_flat, n_bt):
    total = idx_flat.shape[0]
    assert total % (_NW * _CHUNK) == 0
    per_w = total // _NW
    n_chunks = per_w // _CHUNK
    n_vec = per_w // _LANES

    mesh = plsc.VectorSubcoreMesh(core_axis_name="c", subcore_axis_name="s")

    @functools.partial(
        pl.kernel,
        out_type=jax.ShapeDtypeStruct((total,), jnp.float32),
        mesh=mesh,
        scratch_types=[
            pltpu.VMEM((per_w,), jnp.int32),
            pltpu.VMEM((per_w,), jnp.float32),
            pltpu.SemaphoreType.DMA,
            pltpu.SemaphoreType.DMA,
        ],
    )
    def gather_kernel(scores_hbm, idx_hbm, out_hbm, idx_v, vals_v, sem, sem2):
        wid = lax.axis_index("s") * _NC + lax.axis_index("c")
        elem_base = wid * per_w
        half = per_w // 2
        stage2 = pltpu.make_async_copy(
            idx_hbm.at[pl.ds(elem_base + half, half)],
            idx_v.at[pl.ds(half, half)],
            sem2,
        )
        stage2.start()
        pltpu.sync_copy(idx_hbm.at[pl.ds(elem_base, half)], idx_v.at[pl.ds(0, half)])

        lane = lax.iota(jnp.int32, _LANES)
        sh = 10 + (n_bt.bit_length() - 1)
        m_hi = jnp.int32(-(1 << sh))
        m_jr = jnp.int32(7 << 7)
        m_b = jnp.int32(((n_bt - 1) << 10) | 127)

        def chunk_body(c, _):
            col0 = c * _CHUNK

            def vec_body(v, _):
                col = col0 + v * _LANES
                q = lax.broadcast(elem_base + col, (_LANES,)) + lane
                j = idx_v[pl.ds(col, _LANES)]
                hi = lax.bitwise_and(lax.shift_left(j, sh - 3), m_hi)
                jr = lax.bitwise_and(lax.shift_left(j, 7), m_jr)
                idx_v[pl.ds(col, _LANES)] = lax.bitwise_or(
                    lax.bitwise_or(hi, jr), lax.bitwise_and(q, m_b)
                )
                return 0

            lax.fori_loop(0, _CHUNK // _LANES, vec_body, 0)
            pltpu.make_async_copy(
                scores_hbm.at[idx_v.at[pl.ds(col0, _CHUNK)]],
                vals_v.at[pl.ds(col0, _CHUNK)],
                sem,
            ).start()
            return 0

        lax.fori_loop(0, n_chunks // 2, chunk_body, 0)
        stage2.wait()
        lax.fori_loop(n_chunks // 2, n_chunks, chunk_body, 0)
        pltpu.make_async_copy(
            scores_hbm.at[pl.ds(0, per_w)], vals_v, sem
        ).wait()

        pltpu.sync_copy(vals_v, out_hbm.at[pl.ds(elem_base, per_w)])

    return gather_kernel(scores_flat, idx_flat)


def _tc_log_softmax_perm(g4):
    def body(x_ref, o_ref):
        x = x_ref[...]
        m = jnp.max(jnp.max(x, axis=0, keepdims=True), axis=2, keepdims=True)
        e = jnp.exp(x - m)
        s = jnp.sum(jnp.sum(e, axis=0, keepdims=True), axis=2, keepdims=True)
        o_ref[...] = (x - m) - jnp.log(s)

    return pl.pallas_call(
        body,
        out_shape=jax.ShapeDtypeStruct(g4.shape, g4.dtype),
    )(g4)


def kernel(scores, idx):
    b, n = scores.shape
    k = idx.shape[1]
    assert b % 128 == 0 and n % 8 == 0 and k % 8 == 0
    assert (b * k) % (_NW * _CHUNK) == 0
    n_bt = b // 128
    assert n_bt & (n_bt - 1) == 0

    gathered = _sc_gather(_perm_flat(scores), _perm_flat(idx), n_bt)
    out4 = _tc_log_softmax_perm(gathered.reshape(k // 8, n_bt, 8, 128))
    return out4.transpose(1, 3, 0, 2).reshape(b, k)


# speedup vs baseline: 1.0077x; 1.0002x over previous
import functools

import jax
import jax.numpy as jnp
from jax import lax
from jax.experimental import pallas as pl
from jax.experimental.pallas import tpu as pltpu
from jax.experimental.pallas import tpu_sc as plsc

_NC = 2
_NS = 16
_NW = _NC * _NS
_LANES = 16
_CHUNK = 128


def _perm_flat(x):
    b, m = x.shape
    return x.reshape(b // 128, 128, m // 8, 8).transpose(2, 0, 3, 1).reshape(-1)


def _sc_gather(scores_flat, idx_flat, n_bt):
    total = idx_flat.shape[0]
    assert total % (_NW * _CHUNK) == 0
    per_w = total // _NW
    n_chunks = per_w // _CHUNK

    mesh = plsc.VectorSubcoreMesh(core_axis_name="c", subcore_axis_name="s")

    @functools.partial(
        pl.kernel,
        out_type=jax.ShapeDtypeStruct((total,), jnp.float32),
        mesh=mesh,
        scratch_types=[
            pltpu.VMEM((per_w,), jnp.int32),
            pltpu.VMEM((per_w,), jnp.float32),
            pltpu.SemaphoreType.DMA,
            pltpu.SemaphoreType.DMA,
        ],
    )
    def gather_kernel(scores_hbm, idx_hbm, out_hbm, idx_v, vals_v, sem, sem2):
        wid = lax.axis_index("s") * _NC + lax.axis_index("c")
        elem_base = wid * per_w
        half = per_w // 2
        stage2 = pltpu.make_async_copy(
            idx_hbm.at[pl.ds(elem_base + half, half)],
            idx_v.at[pl.ds(half, half)],
            sem2,
        )
        stage2.start()
        pltpu.sync_copy(idx_hbm.at[pl.ds(elem_base, half)], idx_v.at[pl.ds(0, half)])

        lane = lax.iota(jnp.int32, _LANES)
        sh = 10 + (n_bt.bit_length() - 1)
        m_hi = jnp.int32(-(1 << sh))
        m_jr = jnp.int32(7 << 7)
        m_b = jnp.int32(((n_bt - 1) << 10) | 127)

        def chunk_body(c, _):
            col0 = c * _CHUNK

            def vec_body(v, _):
                col = col0 + v * _LANES
                q = lax.broadcast(elem_base + col, (_LANES,)) + lane
                j = idx_v[pl.ds(col, _LANES)]
                hi = lax.bitwise_and(lax.shift_left(j, sh - 3), m_hi)
                jr = lax.bitwise_and(lax.shift_left(j, 7), m_jr)
                idx_v[pl.ds(col, _LANES)] = lax.bitwise_or(
                    lax.bitwise_or(hi, jr), lax.bitwise_and(q, m_b)
                )
                return 0

            lax.fori_loop(0, _CHUNK // _LANES, vec_body, 0)
            pltpu.make_async_copy(
                scores_hbm.at[idx_v.at[pl.ds(col0, _CHUNK)]],
                vals_v.at[pl.ds(col0, _CHUNK)],
                sem,
            ).start()
            return 0

        lax.fori_loop(0, n_chunks // 2, chunk_body, 0)
        stage2.wait()
        lax.fori_loop(n_chunks // 2, n_chunks, chunk_body, 0)
        pltpu.make_async_copy(
            scores_hbm.at[pl.ds(0, per_w)], vals_v, sem
        ).wait()

        pltpu.sync_copy(vals_v, out_hbm.at[pl.ds(elem_base, per_w)])

    return gather_kernel(scores_flat, idx_flat)


def _tc_log_softmax_perm(g4):
    def body(x_ref, o_ref):
        x = x_ref[...]
        m = jnp.max(jnp.max(x, axis=0, keepdims=True), axis=2, keepdims=True)
        e = jnp.exp(x - m)
        s = jnp.sum(jnp.sum(e, axis=0, keepdims=True), axis=2, keepdims=True)
        o_ref[...] = (x - m) - jnp.log(s)

    return pl.pallas_call(
        body,
        out_shape=jax.ShapeDtypeStruct(g4.shape, g4.dtype),
    )(g4)


def kernel(scores, idx):
    b, n = scores.shape
    k = idx.shape[1]
    assert b % 128 == 0 and n % 8 == 0 and k % 8 == 0
    assert (b * k) % (_NW * _CHUNK) == 0
    n_bt = b // 128
    assert n_bt & (n_bt - 1) == 0

    gathered = _sc_gather(_perm_flat(scores), _perm_flat(idx), n_bt)
    out4 = _tc_log_softmax_perm(gathered.reshape(k // 8, n_bt, 8, 128))
    return out4.transpose(1, 3, 0, 2).reshape(b, k)
